# baseline (speedup 1.0000x reference)
import functools

import jax
import jax.numpy as jnp
from jax import lax
from jax.experimental import pallas as pl
from jax.experimental.pallas import tpu as pltpu
from jax.experimental.pallas import tpu_sc as plsc

_NA = 7
_OUT = _NA * 2
_OUTP = 128
_TILE = 128
_NW = 32


def _routing(ids, E, T):
    G = T // _TILE + E
    B = G * _TILE
    eids = jnp.arange(E, dtype=jnp.int32)
    oh = (ids[:, None] == eids[None, :]).astype(jnp.int32)
    counts = oh.sum(axis=0)
    rank = jnp.take_along_axis(jnp.cumsum(oh, axis=0) - oh,
                               ids[:, None], axis=1)[:, 0]
    ntiles = (counts + _TILE - 1) // _TILE
    csum = jnp.cumsum(ntiles)
    step_start = csum - ntiles
    expert_of_step = jnp.clip(
        jnp.searchsorted(csum, jnp.arange(G, dtype=jnp.int32), side="right"),
        0, E - 1).astype(jnp.int32)
    step_valid = (jnp.arange(G, dtype=jnp.int32)
                  < csum[-1]).astype(jnp.int32)
    pos = (step_start[ids] * _TILE + rank).astype(jnp.int32)
    gidx = (jnp.arange(B, dtype=jnp.int32) % T).at[pos].set(
        jnp.arange(T, dtype=jnp.int32))
    return expert_of_step, step_valid, pos, gidx, G, B


def _sc_gather2(x, xi, gidx, B, D):
    bpw = B // _NW
    nchunk = -(-bpw // 128)
    chunk = bpw // nchunk
    assert chunk * nchunk == bpw and chunk % 8 == 0
    ncol = D // 128
    cols = [t[:, c * 128:(c + 1) * 128] for t in (x, xi)
            for c in range(ncol)]
    nt = len(cols)
    mesh = plsc.VectorSubcoreMesh(core_axis_name="c", subcore_axis_name="s")
    col_shape = tuple(jax.ShapeDtypeStruct((B, 128), jnp.float32)
                      for _ in range(nt))

    @functools.partial(
        pl.kernel, mesh=mesh,
        out_type=col_shape,
        scratch_types=[pltpu.VMEM((chunk,), jnp.int32),
                       pltpu.VMEM((chunk,), jnp.int32),
                       pltpu.VMEM((chunk,), jnp.int32),
                       pltpu.VMEM((bpw, 128), jnp.float32),
                       pltpu.VMEM((bpw, 128), jnp.float32),
                       pltpu.SemaphoreType.DMA,
                       pltpu.SemaphoreType.DMA])
    def gk(*refs):
        srcs = refs[:nt]
        idx_hbm = refs[nt]
        outs = refs[nt + 1:2 * nt + 1]
        i0, i1, i2, rows_a, rows_b, gsem, wsem = refs[2 * nt + 1:]
        wid = lax.axis_index("s") * 2 + lax.axis_index("c")
        base = wid * bpw
        idxs = (i0, i1, i2)
        for j in range(nchunk):
            pltpu.sync_copy(
                idx_hbm.at[pl.ds((wid * nchunk + j) * chunk, chunk)],
                idxs[j])
        bufs = (rows_a, rows_b)
        wb = [None, None]
        for t in range(nt):
            buf = bufs[t % 2]
            if wb[t % 2] is not None:
                wb[t % 2].wait()
            copies = [
                pltpu.async_copy(srcs[t].at[idxs[j]],
                                 buf.at[pl.ds(j * chunk, chunk)], gsem)
                for j in range(nchunk)
            ]
            for cp in copies:
                cp.wait()
            wb[t % 2] = pltpu.async_copy(buf, outs[t].at[pl.ds(base, bpw)],
                                         wsem)
        wb[0].wait()
        wb[1].wait()

    return gk(*cols, gidx)


def _sc_unpermute(osort, pos, T):
    bpw = T // _NW
    idx2 = pos.reshape(_NW, bpw)
    mesh = plsc.VectorSubcoreMesh(core_axis_name="c", subcore_axis_name="s")

    @functools.partial(
        pl.kernel, mesh=mesh,
        out_type=jax.ShapeDtypeStruct((T, _OUTP), jnp.float32),
        scratch_types=[pltpu.VMEM((bpw,), jnp.int32),
                       pltpu.VMEM((bpw, _OUTP), jnp.float32),
                       pltpu.SemaphoreType.DMA])
    def uk(src_hbm, idx_hbm, out_hbm, idx_v, rows_v, sem):
        wid = lax.axis_index("s") * 2 + lax.axis_index("c")
        pltpu.sync_copy(idx_hbm.at[wid], idx_v)
        pltpu.async_copy(src_hbm.at[idx_v], rows_v, sem).wait()
        pltpu.sync_copy(rows_v, out_hbm.at[pl.ds(wid * bpw, bpw)])

    return uk(osort, idx2)


def _bdot(a, w):
    return jnp.dot(jnp.maximum(a, 0.0), w,
                   preferred_element_type=jnp.float32)


def _mlp_body(e_sref, v_sref, x0_ref, x1_ref, x2_ref, xi0_ref, xi1_ref,
              xi2_ref, Win_ref, bin_ref, Winit_ref, binit_ref,
              Wb_ref, bb_ref, Wout_ref, bout_ref, out_ref):
    g = pl.program_id(0)

    @pl.when(v_sref[g] == 1)
    def _compute():
        x = jnp.concatenate([x0_ref[...], x1_ref[...], x2_ref[...]], axis=1)
        xi = jnp.concatenate([xi0_ref[...], xi1_ref[...], xi2_ref[...]],
                             axis=1)
        a = _bdot(xi, Winit_ref[0]) + binit_ref[0]
        h = _bdot(x, Win_ref[0]) + bin_ref[0] + a
        for b in range(2):
            t = _bdot(h, Wb_ref[0, 2 * b]) + bb_ref[0, 2 * b]
            t = _bdot(t, Wb_ref[0, 2 * b + 1]) + bb_ref[0, 2 * b + 1]
            h = h + t
        o = _bdot(h, Wout_ref[0]) + bout_ref[0]
        ri = lax.broadcasted_iota(jnp.int32, (_OUTP, _OUTP), 0)
        ci = lax.broadcasted_iota(jnp.int32, (_OUTP, _OUTP), 1)
        pair = (ri // 2 == ci // 2).astype(jnp.float32)
        n = jnp.sqrt(jnp.dot(o * o, pair, preferred_element_type=jnp.float32))
        out_ref[...] = o / jnp.maximum(n, 1e-12)


def _tc_mlp(eos, val, xcols, W_in, bin2, W_init, binit2, Wb4, bb4, Woutp,
            boutp, G, B, C, CH):
    grid_spec = pltpu.PrefetchScalarGridSpec(
        num_scalar_prefetch=2,
        grid=(G,),
        in_specs=[
            pl.BlockSpec((_TILE, 128), lambda g, eref, vref: (g, 0)),
            pl.BlockSpec((_TILE, 128), lambda g, eref, vref: (g, 0)),
            pl.BlockSpec((_TILE, 128), lambda g, eref, vref: (g, 0)),
            pl.BlockSpec((_TILE, 128), lambda g, eref, vref: (g, 0)),
            pl.BlockSpec((_TILE, 128), lambda g, eref, vref: (g, 0)),
            pl.BlockSpec((_TILE, 128), lambda g, eref, vref: (g, 0)),
            pl.BlockSpec((1, C, CH), lambda g, eref, vref: (eref[g], 0, 0)),
            pl.BlockSpec((1, 1, CH), lambda g, eref, vref: (eref[g], 0, 0)),
            pl.BlockSpec((1, C, CH), lambda g, eref, vref: (eref[g], 0, 0)),
            pl.BlockSpec((1, 1, CH), lambda g, eref, vref: (eref[g], 0, 0)),
            pl.BlockSpec((1, 4, CH, CH), lambda g, eref, vref: (eref[g], 0, 0, 0)),
            pl.BlockSpec((1, 4, CH), lambda g, eref, vref: (eref[g], 0, 0)),
            pl.BlockSpec((1, CH, _OUTP), lambda g, eref, vref: (eref[g], 0, 0)),
            pl.BlockSpec((1, 1, _OUTP), lambda g, eref, vref: (eref[g], 0, 0)),
        ],
        out_specs=pl.BlockSpec((_TILE, _OUTP), lambda g, eref, vref: (g, 0)),
    )
    return pl.pallas_call(
        _mlp_body,
        grid_spec=grid_spec,
        out_shape=jax.ShapeDtypeStruct((B, _OUTP), jnp.float32),
        compiler_params=pltpu.CompilerParams(
            dimension_semantics=("arbitrary",)),
    )(eos, val, *xcols, W_in, bin2, W_init, binit2, Wb4, bb4, Woutp, boutp)


def kernel(aa_seqs, s, s_init, W_in, b_in, W_init, b_init, Wb, bb, W_out,
           b_out):
    bs, seq_len, C = s.shape
    T = bs * seq_len
    E, _, CH = W_in.shape

    ids = aa_seqs.reshape(T).astype(jnp.int32)
    sf = s.reshape(T, C)
    sif = s_init.reshape(T, C)

    eos, val, pos, gidx, G, B = _routing(ids, E, T)
    xcols = _sc_gather2(sf, sif, gidx, B, C)

    bin2 = b_in.reshape(E, 1, CH)
    binit2 = b_init.reshape(E, 1, CH)
    Wb4 = Wb.reshape(E, 4, CH, CH)
    bb4 = bb.reshape(E, 4, CH)
    Woutp = jnp.zeros((E, CH, _OUTP), W_out.dtype).at[:, :, :_OUT].set(W_out)
    boutp = jnp.zeros((E, 1, _OUTP), b_out.dtype).at[:, 0, :_OUT].set(b_out)

    osort = _tc_mlp(eos, val, xcols, W_in, bin2, W_init, binit2, Wb4, bb4,
                    Woutp, boutp, G, B, C, CH)
    outp = _sc_unpermute(osort, pos, T)
    return outp[:, :_OUT].reshape(bs, seq_len, _NA, 2)


# speedup vs baseline: 1.1403x; 1.1403x over previous
import functools

import jax
import jax.numpy as jnp
from jax import lax
from jax.experimental import pallas as pl
from jax.experimental.pallas import tpu as pltpu
from jax.experimental.pallas import tpu_sc as plsc

_NA = 7
_OUT = _NA * 2
_OUTP = 128
_TILE = 256
_NW = 32


def _routing(ids, E, T):
    G = T // _TILE + E
    B = G * _TILE
    eids = jnp.arange(E, dtype=jnp.int32)
    oh = (ids[:, None] == eids[None, :]).astype(jnp.int32)
    counts = oh.sum(axis=0)
    rank = jnp.take_along_axis(jnp.cumsum(oh, axis=0) - oh,
                               ids[:, None], axis=1)[:, 0]
    ntiles = (counts + _TILE - 1) // _TILE
    csum = jnp.cumsum(ntiles)
    step_start = csum - ntiles
    expert_of_step = jnp.clip(
        jnp.searchsorted(csum, jnp.arange(G, dtype=jnp.int32), side="right"),
        0, E - 1).astype(jnp.int32)
    step_valid = (jnp.arange(G, dtype=jnp.int32)
                  < csum[-1]).astype(jnp.int32)
    pos = (step_start[ids] * _TILE + rank).astype(jnp.int32)
    gidx = (jnp.arange(B, dtype=jnp.int32) % T).at[pos].set(
        jnp.arange(T, dtype=jnp.int32))
    return expert_of_step, step_valid, pos, gidx, G, B


def _sc_gather2(x, xi, gidx, B, D):
    bpw = B // _NW
    nchunk = -(-bpw // 128)
    chunk = bpw // nchunk
    assert chunk * nchunk == bpw and chunk % 8 == 0
    ncol = D // 128
    cols = [t[:, c * 128:(c + 1) * 128] for t in (x, xi)
            for c in range(ncol)]
    nt = len(cols)
    mesh = plsc.VectorSubcoreMesh(core_axis_name="c", subcore_axis_name="s")
    col_shape = tuple(jax.ShapeDtypeStruct((B, 128), jnp.float32)
                      for _ in range(nt))

    @functools.partial(
        pl.kernel, mesh=mesh,
        out_type=col_shape,
        scratch_types=[pltpu.VMEM((chunk,), jnp.int32),
                       pltpu.VMEM((chunk,), jnp.int32),
                       pltpu.VMEM((chunk,), jnp.int32),
                       pltpu.VMEM((bpw, 128), jnp.float32),
                       pltpu.VMEM((bpw, 128), jnp.float32),
                       pltpu.SemaphoreType.DMA,
                       pltpu.SemaphoreType.DMA])
    def gk(*refs):
        srcs = refs[:nt]
        idx_hbm = refs[nt]
        outs = refs[nt + 1:2 * nt + 1]
        i0, i1, i2, rows_a, rows_b, gsem, wsem = refs[2 * nt + 1:]
        wid = lax.axis_index("s") * 2 + lax.axis_index("c")
        base = wid * bpw
        idxs = (i0, i1, i2)
        for j in range(nchunk):
            pltpu.sync_copy(
                idx_hbm.at[pl.ds((wid * nchunk + j) * chunk, chunk)],
                idxs[j])
        bufs = (rows_a, rows_b)
        wb = [None, None]
        for t in range(nt):
            buf = bufs[t % 2]
            if wb[t % 2] is not None:
                wb[t % 2].wait()
            copies = [
                pltpu.async_copy(srcs[t].at[idxs[j]],
                                 buf.at[pl.ds(j * chunk, chunk)], gsem)
                for j in range(nchunk)
            ]
            for cp in copies:
                cp.wait()
            wb[t % 2] = pltpu.async_copy(buf, outs[t].at[pl.ds(base, bpw)],
                                         wsem)
        wb[0].wait()
        wb[1].wait()

    return gk(*cols, gidx)


def _sc_unpermute(osort, pos, T):
    bpw = T // _NW
    idx2 = pos.reshape(_NW, bpw)
    mesh = plsc.VectorSubcoreMesh(core_axis_name="c", subcore_axis_name="s")

    @functools.partial(
        pl.kernel, mesh=mesh,
        out_type=jax.ShapeDtypeStruct((T, _OUTP), jnp.float32),
        scratch_types=[pltpu.VMEM((bpw,), jnp.int32),
                       pltpu.VMEM((bpw, _OUTP), jnp.float32),
                       pltpu.SemaphoreType.DMA])
    def uk(src_hbm, idx_hbm, out_hbm, idx_v, rows_v, sem):
        wid = lax.axis_index("s") * 2 + lax.axis_index("c")
        pltpu.sync_copy(idx_hbm.at[wid], idx_v)
        pltpu.async_copy(src_hbm.at[idx_v], rows_v, sem).wait()
        pltpu.sync_copy(rows_v, out_hbm.at[pl.ds(wid * bpw, bpw)])

    return uk(osort, idx2)


def _bdot(a, w):
    return jnp.dot(jnp.maximum(a, 0.0), w,
                   preferred_element_type=jnp.float32)


def _mlp_body(e_sref, v_sref, x0_ref, x1_ref, x2_ref, xi0_ref, xi1_ref,
              xi2_ref, Win_ref, bin_ref, Winit_ref, binit_ref,
              Wb_ref, bb_ref, Wout_ref, bout_ref, out_ref):
    g = pl.program_id(0)

    @pl.when(v_sref[g] == 1)
    def _compute():
        x = jnp.concatenate([x0_ref[...], x1_ref[...], x2_ref[...]], axis=1)
        xi = jnp.concatenate([xi0_ref[...], xi1_ref[...], xi2_ref[...]],
                             axis=1)
        a = _bdot(xi, Winit_ref[0]) + binit_ref[0]
        h = _bdot(x, Win_ref[0]) + bin_ref[0] + a
        for b in range(2):
            t = _bdot(h, Wb_ref[0, 2 * b]) + bb_ref[0, 2 * b]
            t = _bdot(t, Wb_ref[0, 2 * b + 1]) + bb_ref[0, 2 * b + 1]
            h = h + t
        o = _bdot(h, Wout_ref[0]) + bout_ref[0]
        ri = lax.broadcasted_iota(jnp.int32, (_OUTP, _OUTP), 0)
        ci = lax.broadcasted_iota(jnp.int32, (_OUTP, _OUTP), 1)
        pair = (ri // 2 == ci // 2).astype(jnp.float32)
        n = jnp.sqrt(jnp.dot(o * o, pair, preferred_element_type=jnp.float32))
        out_ref[...] = o / jnp.maximum(n, 1e-12)


def _tc_mlp(eos, val, xcols, W_in, bin2, W_init, binit2, Wb4, bb4, Woutp,
            boutp, G, B, C, CH):
    grid_spec = pltpu.PrefetchScalarGridSpec(
        num_scalar_prefetch=2,
        grid=(G,),
        in_specs=[
            pl.BlockSpec((_TILE, 128), lambda g, eref, vref: (g, 0)),
            pl.BlockSpec((_TILE, 128), lambda g, eref, vref: (g, 0)),
            pl.BlockSpec((_TILE, 128), lambda g, eref, vref: (g, 0)),
            pl.BlockSpec((_TILE, 128), lambda g, eref, vref: (g, 0)),
            pl.BlockSpec((_TILE, 128), lambda g, eref, vref: (g, 0)),
            pl.BlockSpec((_TILE, 128), lambda g, eref, vref: (g, 0)),
            pl.BlockSpec((1, C, CH), lambda g, eref, vref: (eref[g], 0, 0)),
            pl.BlockSpec((1, 1, CH), lambda g, eref, vref: (eref[g], 0, 0)),
            pl.BlockSpec((1, C, CH), lambda g, eref, vref: (eref[g], 0, 0)),
            pl.BlockSpec((1, 1, CH), lambda g, eref, vref: (eref[g], 0, 0)),
            pl.BlockSpec((1, 4, CH, CH), lambda g, eref, vref: (eref[g], 0, 0, 0)),
            pl.BlockSpec((1, 4, CH), lambda g, eref, vref: (eref[g], 0, 0)),
            pl.BlockSpec((1, CH, _OUTP), lambda g, eref, vref: (eref[g], 0, 0)),
            pl.BlockSpec((1, 1, _OUTP), lambda g, eref, vref: (eref[g], 0, 0)),
        ],
        out_specs=pl.BlockSpec((_TILE, _OUTP), lambda g, eref, vref: (g, 0)),
    )
    return pl.pallas_call(
        _mlp_body,
        grid_spec=grid_spec,
        out_shape=jax.ShapeDtypeStruct((B, _OUTP), jnp.float32),
        compiler_params=pltpu.CompilerParams(
            dimension_semantics=("arbitrary",)),
    )(eos, val, *xcols, W_in, bin2, W_init, binit2, Wb4, bb4, Woutp, boutp)


def kernel(aa_seqs, s, s_init, W_in, b_in, W_init, b_init, Wb, bb, W_out,
           b_out):
    bs, seq_len, C = s.shape
    T = bs * seq_len
    E, _, CH = W_in.shape

    ids = aa_seqs.reshape(T).astype(jnp.int32)
    sf = s.reshape(T, C)
    sif = s_init.reshape(T, C)

    eos, val, pos, gidx, G, B = _routing(ids, E, T)
    xcols = _sc_gather2(sf, sif, gidx, B, C)

    bin2 = b_in.reshape(E, 1, CH)
    binit2 = b_init.reshape(E, 1, CH)
    Wb4 = Wb.reshape(E, 4, CH, CH)
    bb4 = bb.reshape(E, 4, CH)
    Woutp = jnp.zeros((E, CH, _OUTP), W_out.dtype).at[:, :, :_OUT].set(W_out)
    boutp = jnp.zeros((E, 1, _OUTP), b_out.dtype).at[:, 0, :_OUT].set(b_out)

    osort = _tc_mlp(eos, val, xcols, W_in, bin2, W_init, binit2, Wb4, bb4,
                    Woutp, boutp, G, B, C, CH)
    outp = _sc_unpermute(osort, pos, T)
    return outp[:, :_OUT].reshape(bs, seq_len, _NA, 2)
